# baseline (speedup 1.0000x reference)
import functools

import jax
import jax.numpy as jnp
from jax import lax
from jax.experimental import pallas as pl
from jax.experimental.pallas import tpu as pltpu
from jax.experimental.pallas import tpu_sc as plsc

_LEAK = 0.2


def _topk_proj(xt, xT, k, Tn):
    B, N, C = xt.shape
    nT = N // Tn

    def body(xt_t_ref, xT_ref, idx_ref):
        b = pl.program_id(0)
        xt_t = xt_t_ref[0]
        xT_a = xT_ref[0]
        sq_t = jnp.sum(xt_t * xt_t, axis=1, keepdims=True)
        sq_a = jnp.sum(xT_a * xT_a, axis=0, keepdims=True)
        cross = lax.dot_general(
            xt_t.astype(jnp.bfloat16),
            xT_a.astype(jnp.bfloat16),
            (((1,), (0,)), ((), ())),
            preferred_element_type=jnp.float32)
        d = (sq_t + sq_a) - 2.0 * cross
        iota = lax.broadcasted_iota(jnp.int32, (Tn, N), 1)
        cols = []
        for _ in range(k):
            mval = jnp.min(d, axis=1, keepdims=True)
            j = jnp.min(jnp.where(d == mval, iota, N), axis=1, keepdims=True)
            cols.append(j)
            d = jnp.where(iota == j, jnp.float32(jnp.inf), d)
        idx_ref[0] = jnp.concatenate(cols, axis=1) + b * N

    return pl.pallas_call(
        body,
        grid=(B, nT),
        in_specs=[
            pl.BlockSpec((1, Tn, C), lambda b, t: (b, t, 0)),
            pl.BlockSpec((1, C, N), lambda b, t: (b, 0, 0)),
        ],
        out_specs=pl.BlockSpec((1, Tn, k), lambda b, t: (b, t, 0)),
        out_shape=jax.ShapeDtypeStruct((B, N, k), jnp.int32),
        compiler_params=pltpu.CompilerParams(
            dimension_semantics=("parallel", "parallel")),
    )(xt, xT)


def _sc_gather(table, idx):
    R = idx.shape[0]
    V, O = table.shape
    info = plsc.get_sparse_core_info()
    NW = info.num_cores * info.num_subcores
    bpw = R // NW
    chunk = min(bpw, 128)
    nch = bpw // chunk
    mesh = plsc.VectorSubcoreMesh(core_axis_name="c", subcore_axis_name="s")

    @functools.partial(
        pl.kernel,
        mesh=mesh,
        compiler_params=pltpu.CompilerParams(use_tc_tiling_on_sc=False),
        out_type=jax.ShapeDtypeStruct((R, O), jnp.float32),
        scratch_types=[
            pltpu.VMEM((chunk,), jnp.int32),
            pltpu.VMEM((chunk, O), jnp.float32),
            pltpu.SemaphoreType.DMA,
        ],
    )
    def gk(table_hbm, idx_hbm, out_hbm, idx_v, rows_v, sem):
        wid = lax.axis_index("s") * info.num_cores + lax.axis_index("c")
        base = wid * bpw

        def step(c, carry):
            off = base + c * chunk
            pltpu.sync_copy(idx_hbm.at[pl.ds(off, chunk)], idx_v)
            pltpu.async_copy(table_hbm.at[idx_v], rows_v, sem).wait()
            pltpu.sync_copy(rows_v, out_hbm.at[pl.ds(off, chunk)])
            return carry

        lax.fori_loop(0, nch, step, 0)

    return gk(table, idx)


def _conv_max_stats(nbr, xt, W, k, Cpad, Tm):
    B, N, KC = nbr.shape
    C = xt.shape[2]
    O = W.shape[1]
    nT = N // Tm

    def body(g_ref, xt_ref, w_ref, maxh_ref, s_ref, q_ref):
        b = pl.program_id(0)
        t = pl.program_id(1)
        g = g_ref[0]
        x = xt_ref[0]
        wb = w_ref[...].astype(jnp.bfloat16)
        mx = None
        s = None
        sq = None
        for kk in range(k):
            nbrk = g[:, kk * Cpad:kk * Cpad + C]
            feat = jnp.concatenate([nbrk - x, x], axis=1)
            h = lax.dot_general(
                feat.astype(jnp.bfloat16), wb,
                (((1,), (0,)), ((), ())),
                preferred_element_type=jnp.float32)
            if kk == 0:
                mx = h
                s = h
                sq = h * h
            else:
                mx = jnp.maximum(mx, h)
                s = s + h
                sq = sq + h * h
        maxh_ref[0] = mx
        ps = jnp.sum(s, axis=0, keepdims=True)
        pq = jnp.sum(sq, axis=0, keepdims=True)

        @pl.when(jnp.logical_and(b == 0, t == 0))
        def _init():
            s_ref[...] = jnp.zeros(s_ref.shape, s_ref.dtype)
            q_ref[...] = jnp.zeros(q_ref.shape, q_ref.dtype)

        s_ref[...] += ps
        q_ref[...] += pq

    return pl.pallas_call(
        body,
        grid=(B, nT),
        in_specs=[
            pl.BlockSpec((1, Tm, KC), lambda b, t: (b, t, 0)),
            pl.BlockSpec((1, Tm, C), lambda b, t: (b, t, 0)),
            pl.BlockSpec((2 * C, O), lambda b, t: (0, 0)),
        ],
        out_specs=[
            pl.BlockSpec((1, Tm, O), lambda b, t: (b, t, 0)),
            pl.BlockSpec((1, O), lambda b, t: (0, 0)),
            pl.BlockSpec((1, O), lambda b, t: (0, 0)),
        ],
        out_shape=[
            jax.ShapeDtypeStruct((B, N, O), jnp.float32),
            jax.ShapeDtypeStruct((1, O), jnp.float32),
            jax.ShapeDtypeStruct((1, O), jnp.float32),
        ],
    )(nbr, xt, W)


def _norm_leaky(x, scale, shift):
    B, N, O = x.shape
    Tn = min(N, 512)
    nT = N // Tn

    def body(x_ref, sc_ref, sh_ref, o_ref):
        v = x_ref[0] * sc_ref[...] + sh_ref[...]
        o_ref[0] = jnp.where(v > 0, v, _LEAK * v)

    return pl.pallas_call(
        body,
        grid=(B, nT),
        in_specs=[
            pl.BlockSpec((1, Tn, O), lambda b, t: (b, t, 0)),
            pl.BlockSpec((1, O), lambda b, t: (0, 0)),
            pl.BlockSpec((1, O), lambda b, t: (0, 0)),
        ],
        out_specs=pl.BlockSpec((1, Tn, O), lambda b, t: (b, t, 0)),
        out_shape=jax.ShapeDtypeStruct((B, N, O), jnp.float32),
        compiler_params=pltpu.CompilerParams(
            dimension_semantics=("parallel", "parallel")),
    )(x, scale, shift)


def _fps(xc, m):
    _, _, Bh, N = xc.shape
    B = 2 * Bh

    def body(x_ref, idx_ref):
        c = pl.program_id(0)
        x0 = x_ref[0, 0]
        x1 = x_ref[0, 1]
        x2 = x_ref[0, 2]
        iota_n = lax.broadcasted_iota(jnp.int32, (Bh, N), 1)
        iota_m = lax.broadcasted_iota(jnp.int32, (Bh, m), 1)
        rowbase = (lax.broadcasted_iota(jnp.int32, (Bh, 1), 0) + c * Bh) * N
        dists0 = jnp.full((Bh, N), jnp.inf, jnp.float32)
        idx0 = jnp.broadcast_to(rowbase, (Bh, m))
        last0 = jnp.zeros((Bh, 1), jnp.int32)

        def step(i, st):
            dists, idxa, last = st
            sel = iota_n == last
            p0 = jnp.sum(jnp.where(sel, x0, 0.0), axis=1, keepdims=True)
            p1 = jnp.sum(jnp.where(sel, x1, 0.0), axis=1, keepdims=True)
            p2 = jnp.sum(jnp.where(sel, x2, 0.0), axis=1, keepdims=True)
            d0 = x0 - p0
            d1 = x1 - p1
            d2 = x2 - p2
            d = d0 * d0 + d1 * d1 + d2 * d2
            dists = jnp.minimum(dists, d)
            mval = jnp.max(dists, axis=1, keepdims=True)
            nxt = jnp.min(jnp.where(dists == mval, iota_n, N), axis=1,
                          keepdims=True)
            idxa = jnp.where(iota_m == i, nxt + rowbase, idxa)
            return dists, idxa, nxt

        _, idxa, _ = lax.fori_loop(1, m, step, (dists0, idx0, last0))
        idx_ref[0] = idxa

    return pl.pallas_call(
        body,
        grid=(2,),
        in_specs=[pl.BlockSpec((1, 3, Bh, N), lambda c: (c, 0, 0, 0))],
        out_specs=pl.BlockSpec((1, Bh, m), lambda c: (c, 0, 0)),
        out_shape=jax.ShapeDtypeStruct((2, Bh, m), jnp.int32),
        compiler_params=pltpu.CompilerParams(
            dimension_semantics=("parallel",)),
    )(xc)


def _edge_conv_stage(xt, xT, W, g, bb, k, Tn):
    B, N, C = xt.shape
    O = W.shape[1]
    Cpad = max(8, C)
    idx = _topk_proj(xt, xT, k, Tn)
    if Cpad != C:
        table = jnp.pad(xt.reshape(B * N, C), ((0, 0), (0, Cpad - C)))
    else:
        table = xt.reshape(B * N, C)
    nbr = _sc_gather(table, idx.reshape(-1))
    maxh, s, q = _conv_max_stats(nbr.reshape(B, N, k * Cpad), xt, W, k,
                                 Cpad, min(N, 256))
    cnt = B * N * k
    mean = s[0] / cnt
    var = q[0] / cnt - mean * mean
    scale = g / jnp.sqrt(var + 1e-5)
    shift = bb - mean * scale
    return maxh, scale.reshape(1, O), shift.reshape(1, O)


def kernel(input, W1, g1, b1, W2, g2, b2, W3, g3, b3):
    B, _, N = input.shape
    m = 512
    xt0 = jnp.transpose(input, (0, 2, 1))

    maxh1, sc1, sh1 = _edge_conv_stage(xt0, input, W1, g1, b1, 16, 256)
    xc = jnp.transpose(input.reshape(2, B // 2, 3, N), (0, 2, 1, 3))
    fidx = _fps(xc, m)
    mh1g = _sc_gather(maxh1.reshape(B * N, 64), fidx.reshape(-1))
    x1t = _norm_leaky(mh1g.reshape(B, m, 64), sc1, sh1)

    x1T = jnp.transpose(x1t, (0, 2, 1))
    maxh2, sc2, sh2 = _edge_conv_stage(x1t, x1T, W2, g2, b2, 8, 512)
    x2t = _norm_leaky(maxh2, sc2, sh2)

    x2T = jnp.transpose(x2t, (0, 2, 1))
    maxh3, sc3, sh3 = _edge_conv_stage(x2t, x2T, W3, g3, b3, 4, 512)
    x3t = _norm_leaky(maxh3, sc3, sh3)

    out = jnp.concatenate([x1t, x2t, x3t], axis=2)
    return jnp.transpose(out, (0, 2, 1))


# speedup vs baseline: 1.1582x; 1.1582x over previous
import functools

import jax
import jax.numpy as jnp
from jax import lax
from jax.experimental import pallas as pl
from jax.experimental.pallas import tpu as pltpu
from jax.experimental.pallas import tpu_sc as plsc

_LEAK = 0.2


def _topk_proj(xt, xT, k, Tn):
    B, N, C = xt.shape
    nT = N // Tn

    def body(xt_t_ref, xT_ref, idx_ref):
        b = pl.program_id(0)
        xt_t = xt_t_ref[0]
        xT_a = xT_ref[0]
        sq_t = jnp.sum(xt_t * xt_t, axis=1, keepdims=True)
        sq_a = jnp.sum(xT_a * xT_a, axis=0, keepdims=True)
        cross = lax.dot_general(
            xt_t.astype(jnp.bfloat16),
            xT_a.astype(jnp.bfloat16),
            (((1,), (0,)), ((), ())),
            preferred_element_type=jnp.float32)
        d = (sq_t + sq_a) - 2.0 * cross
        iota = lax.broadcasted_iota(jnp.int32, (Tn, N), 1)
        cols = []
        for _ in range(k):
            mval = jnp.min(d, axis=1, keepdims=True)
            j = jnp.min(jnp.where(d == mval, iota, N), axis=1, keepdims=True)
            cols.append(j)
            d = jnp.where(iota == j, jnp.float32(jnp.inf), d)
        idx_ref[0] = jnp.concatenate(cols, axis=1) + b * N

    return pl.pallas_call(
        body,
        grid=(B, nT),
        in_specs=[
            pl.BlockSpec((1, Tn, C), lambda b, t: (b, t, 0)),
            pl.BlockSpec((1, C, N), lambda b, t: (b, 0, 0)),
        ],
        out_specs=pl.BlockSpec((1, Tn, k), lambda b, t: (b, t, 0)),
        out_shape=jax.ShapeDtypeStruct((B, N, k), jnp.int32),
    )(xt, xT)


def _sc_gather(table, idx):
    R = idx.shape[0]
    V, O = table.shape
    info = plsc.get_sparse_core_info()
    NW = info.num_cores * info.num_subcores
    bpw = R // NW
    chunk = min(bpw, 128)
    nch = bpw // chunk
    mesh = plsc.VectorSubcoreMesh(core_axis_name="c", subcore_axis_name="s")

    @functools.partial(
        pl.kernel,
        mesh=mesh,
        compiler_params=pltpu.CompilerParams(use_tc_tiling_on_sc=False),
        out_type=jax.ShapeDtypeStruct((R, O), jnp.float32),
        scratch_types=[
            pltpu.VMEM((chunk,), jnp.int32),
            pltpu.VMEM((chunk, O), jnp.float32),
            pltpu.SemaphoreType.DMA,
        ],
    )
    def gk(table_hbm, idx_hbm, out_hbm, idx_v, rows_v, sem):
        wid = lax.axis_index("s") * info.num_cores + lax.axis_index("c")
        base = wid * bpw

        def step(c, carry):
            off = base + c * chunk
            pltpu.sync_copy(idx_hbm.at[pl.ds(off, chunk)], idx_v)
            pltpu.async_copy(table_hbm.at[idx_v], rows_v, sem).wait()
            pltpu.sync_copy(rows_v, out_hbm.at[pl.ds(off, chunk)])
            return carry

        lax.fori_loop(0, nch, step, 0)

    return gk(table, idx)


def _conv_max_stats(nbr, xt, W, k, Cpad, Tm):
    B, N, KC = nbr.shape
    C = xt.shape[2]
    O = W.shape[1]
    nT = N // Tm

    def body(g_ref, xt_ref, w_ref, maxh_ref, s_ref, q_ref):
        b = pl.program_id(0)
        t = pl.program_id(1)
        g = g_ref[0]
        x = xt_ref[0]
        wb = w_ref[...].astype(jnp.bfloat16)
        mx = None
        s = None
        sq = None
        for kk in range(k):
            nbrk = g[:, kk * Cpad:kk * Cpad + C]
            feat = jnp.concatenate([nbrk - x, x], axis=1)
            h = lax.dot_general(
                feat.astype(jnp.bfloat16), wb,
                (((1,), (0,)), ((), ())),
                preferred_element_type=jnp.float32)
            if kk == 0:
                mx = h
                s = h
                sq = h * h
            else:
                mx = jnp.maximum(mx, h)
                s = s + h
                sq = sq + h * h
        maxh_ref[0] = mx
        ps = jnp.sum(s, axis=0, keepdims=True)
        pq = jnp.sum(sq, axis=0, keepdims=True)

        @pl.when(jnp.logical_and(b == 0, t == 0))
        def _init():
            s_ref[...] = jnp.zeros(s_ref.shape, s_ref.dtype)
            q_ref[...] = jnp.zeros(q_ref.shape, q_ref.dtype)

        s_ref[...] += ps
        q_ref[...] += pq

    return pl.pallas_call(
        body,
        grid=(B, nT),
        in_specs=[
            pl.BlockSpec((1, Tm, KC), lambda b, t: (b, t, 0)),
            pl.BlockSpec((1, Tm, C), lambda b, t: (b, t, 0)),
            pl.BlockSpec((2 * C, O), lambda b, t: (0, 0)),
        ],
        out_specs=[
            pl.BlockSpec((1, Tm, O), lambda b, t: (b, t, 0)),
            pl.BlockSpec((1, O), lambda b, t: (0, 0)),
            pl.BlockSpec((1, O), lambda b, t: (0, 0)),
        ],
        out_shape=[
            jax.ShapeDtypeStruct((B, N, O), jnp.float32),
            jax.ShapeDtypeStruct((1, O), jnp.float32),
            jax.ShapeDtypeStruct((1, O), jnp.float32),
        ],
    )(nbr, xt, W)


def _norm_leaky(x, scale, shift):
    B, N, O = x.shape
    Tn = min(N, 512)
    nT = N // Tn

    def body(x_ref, sc_ref, sh_ref, o_ref):
        v = x_ref[0] * sc_ref[...] + sh_ref[...]
        o_ref[0] = jnp.where(v > 0, v, _LEAK * v)

    return pl.pallas_call(
        body,
        grid=(B, nT),
        in_specs=[
            pl.BlockSpec((1, Tn, O), lambda b, t: (b, t, 0)),
            pl.BlockSpec((1, O), lambda b, t: (0, 0)),
            pl.BlockSpec((1, O), lambda b, t: (0, 0)),
        ],
        out_specs=pl.BlockSpec((1, Tn, O), lambda b, t: (b, t, 0)),
        out_shape=jax.ShapeDtypeStruct((B, N, O), jnp.float32),
    )(x, scale, shift)


def _fps(xc, m):
    _, B, N = xc.shape

    def body(x_ref, idx_ref):
        x0 = x_ref[0]
        x1 = x_ref[1]
        x2 = x_ref[2]
        iota_n = lax.broadcasted_iota(jnp.int32, (B, N), 1)
        iota_m = lax.broadcasted_iota(jnp.int32, (B, m), 1)
        rowbase = lax.broadcasted_iota(jnp.int32, (B, 1), 0) * N
        dists0 = jnp.full((B, N), jnp.inf, jnp.float32)
        idx0 = jnp.broadcast_to(rowbase, (B, m))
        last0 = jnp.zeros((B, 1), jnp.int32)

        def step(i, st):
            dists, idxa, last = st
            sel = iota_n == last
            p0 = jnp.sum(jnp.where(sel, x0, 0.0), axis=1, keepdims=True)
            p1 = jnp.sum(jnp.where(sel, x1, 0.0), axis=1, keepdims=True)
            p2 = jnp.sum(jnp.where(sel, x2, 0.0), axis=1, keepdims=True)
            d0 = x0 - p0
            d1 = x1 - p1
            d2 = x2 - p2
            d = d0 * d0 + d1 * d1 + d2 * d2
            dists = jnp.minimum(dists, d)
            mval = jnp.max(dists, axis=1, keepdims=True)
            nxt = jnp.min(jnp.where(dists == mval, iota_n, N), axis=1,
                          keepdims=True)
            idxa = jnp.where(iota_m == i, nxt + rowbase, idxa)
            return dists, idxa, nxt

        _, idxa, _ = lax.fori_loop(1, m, step, (dists0, idx0, last0))
        idx_ref[...] = idxa

    return pl.pallas_call(
        body,
        in_specs=[pl.BlockSpec((3, B, N), lambda: (0, 0, 0))],
        out_specs=pl.BlockSpec((B, m), lambda: (0, 0)),
        out_shape=jax.ShapeDtypeStruct((B, m), jnp.int32),
    )(xc)


def _edge_conv_stage(xt, xT, W, g, bb, k, Tn):
    B, N, C = xt.shape
    O = W.shape[1]
    Cpad = max(8, C)
    idx = _topk_proj(xt, xT, k, Tn)
    if Cpad != C:
        table = jnp.pad(xt.reshape(B * N, C), ((0, 0), (0, Cpad - C)))
    else:
        table = xt.reshape(B * N, C)
    nbr = _sc_gather(table, idx.reshape(-1))
    maxh, s, q = _conv_max_stats(nbr.reshape(B, N, k * Cpad), xt, W, k,
                                 Cpad, min(N, 256))
    cnt = B * N * k
    mean = s[0] / cnt
    var = q[0] / cnt - mean * mean
    scale = g / jnp.sqrt(var + 1e-5)
    shift = bb - mean * scale
    return maxh, scale.reshape(1, O), shift.reshape(1, O)


def kernel(input, W1, g1, b1, W2, g2, b2, W3, g3, b3):
    B, _, N = input.shape
    m = 512
    xt0 = jnp.transpose(input, (0, 2, 1))

    maxh1, sc1, sh1 = _edge_conv_stage(xt0, input, W1, g1, b1, 16, 256)
    fidx = _fps(jnp.transpose(input, (1, 0, 2)), m)
    mh1g = _sc_gather(maxh1.reshape(B * N, 64), fidx.reshape(-1))
    x1t = _norm_leaky(mh1g.reshape(B, m, 64), sc1, sh1)

    x1T = jnp.transpose(x1t, (0, 2, 1))
    maxh2, sc2, sh2 = _edge_conv_stage(x1t, x1T, W2, g2, b2, 8, 512)
    x2t = _norm_leaky(maxh2, sc2, sh2)

    x2T = jnp.transpose(x2t, (0, 2, 1))
    maxh3, sc3, sh3 = _edge_conv_stage(x2t, x2T, W3, g3, b3, 4, 512)
    x3t = _norm_leaky(maxh3, sc3, sh3)

    out = jnp.concatenate([x1t, x2t, x3t], axis=2)
    return jnp.transpose(out, (0, 2, 1))
